# baseline (speedup 1.0000x reference)
import jax
import jax.numpy as jnp
from jax.experimental import pallas as pl
from jax.experimental.pallas import tpu as pltpu

_BT = 65536
_DEPTH = 3


def kernel(sample):
    B, C, T = sample.shape
    nch = T // _BT

    def body(x_hbm, o_hbm, in_buf, out_buf, sem_in, sem_out):
        def in_copy(k, slot):
            return pltpu.make_async_copy(
                x_hbm.at[:, :, pl.ds(k * _BT, _BT)], in_buf.at[slot], sem_in.at[slot]
            )

        def out_copy(k, slot):
            return pltpu.make_async_copy(
                out_buf.at[slot], o_hbm.at[:, :, pl.ds(k * _BT, _BT)], sem_out.at[slot]
            )

        for k in range(min(_DEPTH, nch)):
            in_copy(k, k).start()

        for k in range(nch):
            slot = k % _DEPTH
            in_copy(k, slot).wait()
            if k >= _DEPTH:
                out_copy(k - _DEPTH, slot).wait()
            s = in_buf[slot]
            out_buf[slot, :, 0, :] = s[:, 0, :] + s[:, 1, :] + s[:, 2, :]
            out_buf[slot, :, 1, :] = s[:, 3, :]
            out_copy(k, slot).start()
            if k + _DEPTH < nch:
                in_copy(k + _DEPTH, slot).start()

        for k in range(max(nch - _DEPTH, 0), nch):
            out_copy(k, k % _DEPTH).wait()

    return pl.pallas_call(
        body,
        in_specs=[pl.BlockSpec(memory_space=pltpu.HBM)],
        out_specs=pl.BlockSpec(memory_space=pltpu.HBM),
        out_shape=jax.ShapeDtypeStruct((B, 2, T), sample.dtype),
        scratch_shapes=[
            pltpu.VMEM((_DEPTH, B, C, _BT), jnp.float32),
            pltpu.VMEM((_DEPTH, B, 2, _BT), jnp.float32),
            pltpu.SemaphoreType.DMA((_DEPTH,)),
            pltpu.SemaphoreType.DMA((_DEPTH,)),
        ],
    )(sample)


# speedup vs baseline: 1.0163x; 1.0163x over previous
import jax
import jax.numpy as jnp
from jax.experimental import pallas as pl
from jax.experimental.pallas import tpu as pltpu

_BT = 16384
_DEPTH = 4


def kernel(sample):
    B, C, T = sample.shape
    nch = T // _BT

    def body(x_hbm, o_hbm, in_buf, out_buf, sem_in, sem_out):
        def in_copy(k, slot):
            return pltpu.make_async_copy(
                x_hbm.at[:, :, pl.ds(k * _BT, _BT)], in_buf.at[slot], sem_in.at[slot]
            )

        def out_copy(k, slot):
            return pltpu.make_async_copy(
                out_buf.at[slot], o_hbm.at[:, :, pl.ds(k * _BT, _BT)], sem_out.at[slot]
            )

        for k in range(min(_DEPTH, nch)):
            in_copy(k, k).start()

        for k in range(nch):
            slot = k % _DEPTH
            in_copy(k, slot).wait()
            if k >= _DEPTH:
                out_copy(k - _DEPTH, slot).wait()
            s = in_buf[slot]
            out_buf[slot, :, 0, :] = s[:, 0, :] + s[:, 1, :] + s[:, 2, :]
            out_buf[slot, :, 1, :] = s[:, 3, :]
            out_copy(k, slot).start()
            if k + _DEPTH < nch:
                in_copy(k + _DEPTH, slot).start()

        for k in range(max(nch - _DEPTH, 0), nch):
            out_copy(k, k % _DEPTH).wait()

    return pl.pallas_call(
        body,
        in_specs=[pl.BlockSpec(memory_space=pltpu.HBM)],
        out_specs=pl.BlockSpec(memory_space=pltpu.HBM),
        out_shape=jax.ShapeDtypeStruct((B, 2, T), sample.dtype),
        scratch_shapes=[
            pltpu.VMEM((_DEPTH, B, C, _BT), jnp.float32),
            pltpu.VMEM((_DEPTH, B, 2, _BT), jnp.float32),
            pltpu.SemaphoreType.DMA((_DEPTH,)),
            pltpu.SemaphoreType.DMA((_DEPTH,)),
        ],
    )(sample)


# speedup vs baseline: 1.0227x; 1.0063x over previous
import jax
import jax.numpy as jnp
from jax.experimental import pallas as pl
from jax.experimental.pallas import tpu as pltpu

_BT = 8192
_DEPTH = 6


def kernel(sample):
    B, C, T = sample.shape
    nch = T // _BT

    def body(x_hbm, o_hbm, in_buf, out_buf, sem_in, sem_out):
        def in_copy(k, slot):
            return pltpu.make_async_copy(
                x_hbm.at[:, :, pl.ds(k * _BT, _BT)], in_buf.at[slot], sem_in.at[slot]
            )

        def out_copy(k, slot):
            return pltpu.make_async_copy(
                out_buf.at[slot], o_hbm.at[:, :, pl.ds(k * _BT, _BT)], sem_out.at[slot]
            )

        for k in range(min(_DEPTH, nch)):
            in_copy(k, k).start()

        for k in range(nch):
            slot = k % _DEPTH
            in_copy(k, slot).wait()
            if k >= _DEPTH:
                out_copy(k - _DEPTH, slot).wait()
            s = in_buf[slot]
            out_buf[slot, :, 0, :] = s[:, 0, :] + s[:, 1, :] + s[:, 2, :]
            out_buf[slot, :, 1, :] = s[:, 3, :]
            out_copy(k, slot).start()
            if k + _DEPTH < nch:
                in_copy(k + _DEPTH, slot).start()

        for k in range(max(nch - _DEPTH, 0), nch):
            out_copy(k, k % _DEPTH).wait()

    return pl.pallas_call(
        body,
        in_specs=[pl.BlockSpec(memory_space=pltpu.HBM)],
        out_specs=pl.BlockSpec(memory_space=pltpu.HBM),
        out_shape=jax.ShapeDtypeStruct((B, 2, T), sample.dtype),
        scratch_shapes=[
            pltpu.VMEM((_DEPTH, B, C, _BT), jnp.float32),
            pltpu.VMEM((_DEPTH, B, 2, _BT), jnp.float32),
            pltpu.SemaphoreType.DMA((_DEPTH,)),
            pltpu.SemaphoreType.DMA((_DEPTH,)),
        ],
    )(sample)
